# Initial kernel scaffold; baseline (speedup 1.0000x reference)
import jax
import jax.numpy as jnp
from jax import lax
from jax.experimental import pallas as pl
from jax.experimental.pallas import tpu as pltpu
from jax.experimental.pallas import tpu_sc as plsc

_BLK = 4096

_NC = 2
_NS = 16
_L = 16
_NW = _NC * _NS


def _mlp_body(x_ref, w1t_ref, b1_ref, w2at_ref, w2bt_ref, b2_ref, o_ref):
    x = x_ref[...]
    featst = lax.dot_general(
        w1t_ref[...], x, (((1,), (1,)), ((), ())),
        preferred_element_type=jnp.float32)
    featst = jnp.maximum(featst + b1_ref[...], 0.0)
    out2t = lax.dot_general(
        w2at_ref[...], featst, (((1,), (0,)), ((), ())),
        preferred_element_type=jnp.float32)
    out2t = out2t + lax.dot_general(
        w2bt_ref[0:4, :], x, (((1,), (1,)), ((), ())),
        preferred_element_type=jnp.float32)
    out2t = out2t + b2_ref[...]
    deltat = lax.dot_general(
        w2bt_ref[4:5, :], x, (((1,), (1,)), ((), ())),
        preferred_element_type=jnp.float32)
    st = jnp.maximum(out2t[0:1, :], 0.0) * deltat
    rgbt = jax.nn.sigmoid(out2t[1:4, :])
    o_ref[0:1, :] = st
    o_ref[1:4, :] = rgbt


def _run_mlp(packed, w1t, b1, w2at, w2bt, b2):
    n = packed.shape[0]
    grid = n // _BLK
    return pl.pallas_call(
        _mlp_body,
        grid=(grid,),
        in_specs=[
            pl.BlockSpec((_BLK, 7), lambda i: (i, 0)),
            pl.BlockSpec((32, 7), lambda i: (0, 0)),
            pl.BlockSpec((32, 1), lambda i: (0, 0)),
            pl.BlockSpec((4, 32), lambda i: (0, 0)),
            pl.BlockSpec((5, 7), lambda i: (0, 0)),
            pl.BlockSpec((4, 1), lambda i: (0, 0)),
        ],
        out_specs=pl.BlockSpec((4, _BLK), lambda i: (0, i)),
        out_shape=jax.ShapeDtypeStruct((4, n), jnp.float32),
    )(packed, w1t, b1, w2at, w2bt, b2)


def _render_body(mlp_hbm, bg_hbm, out_hbm, s_v, r_v, g_v, b_v, bg_v, out_v):
    wid = lax.axis_index("s") * _NC + lax.axis_index("c")
    spw = (8192 // _NW) * 64
    base = wid * spw
    pltpu.sync_copy(mlp_hbm.at[0, pl.ds(base, spw)], s_v)
    pltpu.sync_copy(mlp_hbm.at[1, pl.ds(base, spw)], r_v)
    pltpu.sync_copy(mlp_hbm.at[2, pl.ds(base, spw)], g_v)
    pltpu.sync_copy(mlp_hbm.at[3, pl.ds(base, spw)], b_v)
    pltpu.sync_copy(bg_hbm, bg_v)

    lanes = lax.iota(jnp.int32, (16,))
    ray_word = lanes * 64

    def step(i, carry):
        idx, t, ar, ag, ab, aw = carry
        s = plsc.load_gather(s_v, [idx])
        r = plsc.load_gather(r_v, [idx])
        g = plsc.load_gather(g_v, [idx])
        b = plsc.load_gather(b_v, [idx])
        e = jnp.exp(-s)
        w = (1.0 - e) * t
        return (idx + 1, t * e,
                ar + w * r, ag + w * g, ab + w * b, aw + w)

    for gi in range(8192 // _NW // 16):
        idx0 = ray_word + gi * (16 * 64)
        ones = jnp.ones((16,), jnp.float32)
        zeros = jnp.zeros((16,), jnp.float32)
        _, _, ar, ag, ab, aw = lax.fori_loop(
            0, 64, step, (idx0, ones, zeros, zeros, zeros, zeros))

        rem = 1.0 - aw
        orow = (gi * 16 + lanes) * 3
        plsc.store_scatter(out_v, [orow], ar + bg_v[pl.ds(0, 16)] * rem)
        plsc.store_scatter(out_v, [orow + 1], ag + bg_v[pl.ds(16, 16)] * rem)
        plsc.store_scatter(out_v, [orow + 2], ab + bg_v[pl.ds(32, 16)] * rem)

    rpw = 8192 // _NW
    pltpu.sync_copy(out_v, out_hbm.at[pl.ds(wid * rpw * 3, rpw * 3)])


def _run_render(mlp, bg48, n_rays):
    mesh = plsc.VectorSubcoreMesh(core_axis_name="c", subcore_axis_name="s")
    spw = (n_rays // _NW) * 64
    rpw = n_rays // _NW
    kern = pl.kernel(
        _render_body,
        out_type=jax.ShapeDtypeStruct((n_rays * 3,), jnp.float32),
        mesh=mesh,
        scratch_types=[
            pltpu.VMEM((spw,), jnp.float32),
            pltpu.VMEM((spw,), jnp.float32),
            pltpu.VMEM((spw,), jnp.float32),
            pltpu.VMEM((spw,), jnp.float32),
            pltpu.VMEM((48,), jnp.float32),
            pltpu.VMEM((rpw * 3,), jnp.float32),
        ],
    )
    return kern(mlp, bg48)


@jax.jit
def kernel(packed_samples, packing_info, W_feat, b_feat, W_sigma, b_sigma,
           W_rgb, b_rgb, bg_color):
    n_rays = packing_info.shape[0]
    w1t = jnp.concatenate(
        [W_feat.T, jnp.zeros((32, 4), jnp.float32)], axis=1)
    b1 = b_feat[:, None]
    w2at = jnp.concatenate([W_sigma, W_rgb[:32]], axis=1).T
    w2b = jnp.zeros((5, 7), jnp.float32)
    w2b = w2b.at[1:4, 3:6].set(W_rgb[32:35].T)
    w2b = w2b.at[4, 6].set(1.0)
    b2 = jnp.concatenate([b_sigma, b_rgb])[:, None]

    mlp = _run_mlp(packed_samples, w1t, b1, w2at, w2b, b2)

    bg48 = jnp.repeat(bg_color, 16)
    out = _run_render(mlp, bg48, n_rays)
    return out.reshape(n_rays, 3)


# speedup vs baseline: 5.5198x; 5.5198x over previous
import jax
import jax.numpy as jnp
from jax import lax
from jax.experimental import pallas as pl
from jax.experimental.pallas import tpu as pltpu
from jax.experimental.pallas import tpu_sc as plsc

_BLK = 8192

_NC = 2
_NS = 16
_NW = _NC * _NS

_SC_PARAMS = pltpu.CompilerParams(needs_layout_passes=False)


def _transpose_body(x_hbm, xt_hbm, in_v, out_v):
    wid = lax.axis_index("s") * _NC + lax.axis_index("c")
    n = x_hbm.shape[0] // 7
    spw = n // _NW
    sub = 4096
    lanes = lax.iota(jnp.int32, 16)
    for k in range(spw // sub):
        base = wid * spw + k * sub
        pltpu.sync_copy(x_hbm.at[pl.ds(base * 7, sub * 7)], in_v)

        def step(v, idx):
            off = v * 16
            for j in range(7):
                col = plsc.load_gather(in_v, [idx + j])
                out_v[j, pl.ds(off, 16)] = col
            return idx + 16 * 7

        lax.fori_loop(0, sub // 16, step, lanes * 7)
        pltpu.sync_copy(out_v, xt_hbm.at[:, pl.ds(base, sub)])


def _run_transpose(packed_flat, n):
    mesh = plsc.VectorSubcoreMesh(core_axis_name="c", subcore_axis_name="s")
    kern = pl.kernel(
        _transpose_body,
        out_type=jax.ShapeDtypeStruct((7, n), jnp.float32),
        mesh=mesh,
        scratch_types=[
            pltpu.VMEM((4096 * 7,), jnp.float32),
            pltpu.VMEM((7, 4096), jnp.float32),
        ],
        compiler_params=_SC_PARAMS,
    )
    return kern(packed_flat)


def _mlp_body(xt_ref, w1t_ref, b1_ref, w2at_ref, w2bt_ref, b2_ref, o_ref):
    xt = xt_ref[...]
    featst = lax.dot_general(
        w1t_ref[...], xt, (((1,), (0,)), ((), ())),
        preferred_element_type=jnp.float32)
    featst = jnp.maximum(featst + b1_ref[...], 0.0)
    out2t = lax.dot_general(
        w2at_ref[...], featst, (((1,), (0,)), ((), ())),
        preferred_element_type=jnp.float32)
    out2t = out2t + lax.dot_general(
        w2bt_ref[...], xt, (((1,), (0,)), ((), ())),
        preferred_element_type=jnp.float32)
    out2t = out2t + b2_ref[...]
    st = jnp.maximum(out2t[0:1, :], 0.0) * xt[6:7, :]
    rgbt = jax.nn.sigmoid(out2t[1:4, :])
    o_ref[0:1, :] = st
    o_ref[1:4, :] = rgbt


def _run_mlp(xt, w1t, b1, w2at, w2bt, b2):
    n = xt.shape[1]
    grid = n // _BLK
    return pl.pallas_call(
        _mlp_body,
        grid=(grid,),
        in_specs=[
            pl.BlockSpec((7, _BLK), lambda i: (0, i)),
            pl.BlockSpec((32, 7), lambda i: (0, 0)),
            pl.BlockSpec((32, 1), lambda i: (0, 0)),
            pl.BlockSpec((4, 32), lambda i: (0, 0)),
            pl.BlockSpec((4, 7), lambda i: (0, 0)),
            pl.BlockSpec((4, 1), lambda i: (0, 0)),
        ],
        out_specs=pl.BlockSpec((4, _BLK), lambda i: (0, i)),
        out_shape=jax.ShapeDtypeStruct((4, n), jnp.float32),
    )(xt, w1t, b1, w2at, w2bt, b2)


def _render_body(mlp_hbm, bg_hbm, out_hbm, s_v, r_v, g_v, b_v, bg_v, out_v):
    wid = lax.axis_index("s") * _NC + lax.axis_index("c")
    spw = (8192 // _NW) * 64
    base = wid * spw
    pltpu.sync_copy(mlp_hbm.at[0, pl.ds(base, spw)], s_v)
    pltpu.sync_copy(mlp_hbm.at[1, pl.ds(base, spw)], r_v)
    pltpu.sync_copy(mlp_hbm.at[2, pl.ds(base, spw)], g_v)
    pltpu.sync_copy(mlp_hbm.at[3, pl.ds(base, spw)], b_v)
    pltpu.sync_copy(bg_hbm, bg_v)

    lanes = lax.iota(jnp.int32, 16)
    ray_word = lanes * 64

    def step(i, carry):
        idx, t, ar, ag, ab, aw = carry
        s = plsc.load_gather(s_v, [idx])
        r = plsc.load_gather(r_v, [idx])
        g = plsc.load_gather(g_v, [idx])
        b = plsc.load_gather(b_v, [idx])
        e = jnp.exp(-s)
        w = (1.0 - e) * t
        return (idx + 1, t * e,
                ar + w * r, ag + w * g, ab + w * b, aw + w)

    for gi in range(8192 // _NW // 16):
        idx0 = ray_word + gi * (16 * 64)
        ones = jnp.ones((16,), jnp.float32)
        zeros = jnp.zeros((16,), jnp.float32)
        _, _, ar, ag, ab, aw = lax.fori_loop(
            0, 64, step, (idx0, ones, zeros, zeros, zeros, zeros))

        rem = 1.0 - aw
        orow = (gi * 16 + lanes) * 3
        plsc.store_scatter(out_v, [orow], ar + bg_v[pl.ds(0, 16)] * rem)
        plsc.store_scatter(out_v, [orow + 1], ag + bg_v[pl.ds(16, 16)] * rem)
        plsc.store_scatter(out_v, [orow + 2], ab + bg_v[pl.ds(32, 16)] * rem)

    rpw = 8192 // _NW
    pltpu.sync_copy(out_v, out_hbm.at[pl.ds(wid * rpw * 3, rpw * 3)])


def _run_render(mlp, bg48, n_rays):
    mesh = plsc.VectorSubcoreMesh(core_axis_name="c", subcore_axis_name="s")
    spw = (n_rays // _NW) * 64
    rpw = n_rays // _NW
    kern = pl.kernel(
        _render_body,
        out_type=jax.ShapeDtypeStruct((n_rays * 3,), jnp.float32),
        mesh=mesh,
        scratch_types=[
            pltpu.VMEM((spw,), jnp.float32),
            pltpu.VMEM((spw,), jnp.float32),
            pltpu.VMEM((spw,), jnp.float32),
            pltpu.VMEM((spw,), jnp.float32),
            pltpu.VMEM((48,), jnp.float32),
            pltpu.VMEM((rpw * 3,), jnp.float32),
        ],
        compiler_params=_SC_PARAMS,
    )
    return kern(mlp, bg48)


@jax.jit
def kernel(packed_samples, packing_info, W_feat, b_feat, W_sigma, b_sigma,
           W_rgb, b_rgb, bg_color):
    n_rays = packing_info.shape[0]
    n = packed_samples.shape[0]
    w1t = jnp.concatenate(
        [W_feat.T, jnp.zeros((32, 4), jnp.float32)], axis=1)
    b1 = b_feat[:, None]
    w2at = jnp.concatenate([W_sigma, W_rgb[:32]], axis=1).T
    w2bt = jnp.zeros((4, 7), jnp.float32)
    w2bt = w2bt.at[1:4, 3:6].set(W_rgb[32:35].T)
    b2 = jnp.concatenate([b_sigma, b_rgb])[:, None]

    xt = _run_transpose(packed_samples.reshape(-1), n)
    mlp = _run_mlp(xt, w1t, b1, w2at, w2bt, b2)

    bg48 = jnp.repeat(bg_color, 16)
    out = _run_render(mlp, bg48, n_rays)
    return out.reshape(n_rays, 3)


# speedup vs baseline: 6.4458x; 1.1678x over previous
import jax
import jax.numpy as jnp
from jax import lax
from jax.experimental import pallas as pl
from jax.experimental.pallas import tpu as pltpu
from jax.experimental.pallas import tpu_sc as plsc

_BLK = 8192

_NC = 2
_NS = 16
_NW = _NC * _NS

_SC_PARAMS = pltpu.CompilerParams(needs_layout_passes=False)


def _mlp_body(x_ref, w1t_ref, b1_ref, w2at_ref, w2bt_ref, b2_ref,
              s_ref, r_ref, g_ref, b_ref):
    x = x_ref[...]
    featst = lax.dot_general(
        w1t_ref[...], x, (((1,), (1,)), ((), ())),
        preferred_element_type=jnp.float32)
    featst = jnp.maximum(featst + b1_ref[...], 0.0)
    out2t = lax.dot_general(
        w2at_ref[...], featst, (((1,), (0,)), ((), ())),
        preferred_element_type=jnp.float32)
    out2t = out2t + lax.dot_general(
        w2bt_ref[0:4, :], x, (((1,), (1,)), ((), ())),
        preferred_element_type=jnp.float32)
    out2t = out2t + b2_ref[...]
    deltat = lax.dot_general(
        w2bt_ref[4:5, :], x, (((1,), (1,)), ((), ())),
        preferred_element_type=jnp.float32)
    st = jnp.maximum(out2t[0:1, :], 0.0) * deltat
    rgbt = jax.nn.sigmoid(out2t[1:4, :])
    s_ref[...] = st.reshape(_BLK)
    r_ref[...] = rgbt[0:1, :].reshape(_BLK)
    g_ref[...] = rgbt[1:2, :].reshape(_BLK)
    b_ref[...] = rgbt[2:3, :].reshape(_BLK)


def _run_mlp(packed, w1t, b1, w2at, w2bt, b2):
    n = packed.shape[0]
    grid = n // _BLK
    vec = jax.ShapeDtypeStruct((n,), jnp.float32)
    return pl.pallas_call(
        _mlp_body,
        grid=(grid,),
        in_specs=[
            pl.BlockSpec((_BLK, 7), lambda i: (i, 0)),
            pl.BlockSpec((32, 7), lambda i: (0, 0)),
            pl.BlockSpec((32, 1), lambda i: (0, 0)),
            pl.BlockSpec((4, 32), lambda i: (0, 0)),
            pl.BlockSpec((5, 7), lambda i: (0, 0)),
            pl.BlockSpec((4, 1), lambda i: (0, 0)),
        ],
        out_specs=[pl.BlockSpec((_BLK,), lambda i: (i,))] * 4,
        out_shape=[vec, vec, vec, vec],
    )(packed, w1t, b1, w2at, w2bt, b2)


def _render_body(s_hbm, r_hbm, g_hbm, b_hbm, bg_hbm, out_hbm,
                 s_v, r_v, g_v, b_v, bg_v, out_v):
    wid = lax.axis_index("s") * _NC + lax.axis_index("c")
    spw = (8192 // _NW) * 64
    base = wid * spw
    pltpu.sync_copy(s_hbm.at[pl.ds(base, spw)], s_v)
    pltpu.sync_copy(r_hbm.at[pl.ds(base, spw)], r_v)
    pltpu.sync_copy(g_hbm.at[pl.ds(base, spw)], g_v)
    pltpu.sync_copy(b_hbm.at[pl.ds(base, spw)], b_v)
    pltpu.sync_copy(bg_hbm, bg_v)

    lanes = lax.iota(jnp.int32, 16)
    ray_word = lanes * 64

    def quad(i, carry):
        idx, t, ar, ag, ab, aw = carry
        for _ in range(4):
            s = plsc.load_gather(s_v, [idx])
            r = plsc.load_gather(r_v, [idx])
            g = plsc.load_gather(g_v, [idx])
            b = plsc.load_gather(b_v, [idx])
            e = jnp.exp(-s)
            w = (1.0 - e) * t
            t = t * e
            ar = ar + w * r
            ag = ag + w * g
            ab = ab + w * b
            aw = aw + w
            idx = idx + 1
        return (idx, t, ar, ag, ab, aw)

    for gi in range(8192 // _NW // 16):
        idx0 = ray_word + gi * (16 * 64)
        ones = jnp.ones((16,), jnp.float32)
        zeros = jnp.zeros((16,), jnp.float32)
        _, _, ar, ag, ab, aw = lax.fori_loop(
            0, 16, quad, (idx0, ones, zeros, zeros, zeros, zeros))

        rem = 1.0 - aw
        orow = (gi * 16 + lanes) * 3
        plsc.store_scatter(out_v, [orow], ar + bg_v[pl.ds(0, 16)] * rem)
        plsc.store_scatter(out_v, [orow + 1], ag + bg_v[pl.ds(16, 16)] * rem)
        plsc.store_scatter(out_v, [orow + 2], ab + bg_v[pl.ds(32, 16)] * rem)

    rpw = 8192 // _NW
    pltpu.sync_copy(out_v, out_hbm.at[pl.ds(wid * rpw * 3, rpw * 3)])


def _run_render(s, r, g, b, bg48, n_rays):
    mesh = plsc.VectorSubcoreMesh(core_axis_name="c", subcore_axis_name="s")
    spw = (n_rays // _NW) * 64
    rpw = n_rays // _NW
    kern = pl.kernel(
        _render_body,
        out_type=jax.ShapeDtypeStruct((n_rays * 3,), jnp.float32),
        mesh=mesh,
        scratch_types=[
            pltpu.VMEM((spw,), jnp.float32),
            pltpu.VMEM((spw,), jnp.float32),
            pltpu.VMEM((spw,), jnp.float32),
            pltpu.VMEM((spw,), jnp.float32),
            pltpu.VMEM((48,), jnp.float32),
            pltpu.VMEM((rpw * 3,), jnp.float32),
        ],
        compiler_params=_SC_PARAMS,
    )
    return kern(s, r, g, b, bg48)


@jax.jit
def kernel(packed_samples, packing_info, W_feat, b_feat, W_sigma, b_sigma,
           W_rgb, b_rgb, bg_color):
    n_rays = packing_info.shape[0]
    w1t = jnp.concatenate(
        [W_feat.T, jnp.zeros((32, 4), jnp.float32)], axis=1)
    b1 = b_feat[:, None]
    w2at = jnp.concatenate([W_sigma, W_rgb[:32]], axis=1).T
    w2bt = jnp.zeros((5, 7), jnp.float32)
    w2bt = w2bt.at[1:4, 3:6].set(W_rgb[32:35].T)
    w2bt = w2bt.at[4, 6].set(1.0)
    b2 = jnp.concatenate([b_sigma, b_rgb])[:, None]

    s, r, g, b = _run_mlp(packed_samples, w1t, b1, w2at, w2bt, b2)

    bg48 = jnp.repeat(bg_color, 16)
    out = _run_render(s, r, g, b, bg48, n_rays)
    return out.reshape(n_rays, 3)


# speedup vs baseline: 17.8382x; 2.7674x over previous
import jax
import jax.numpy as jnp
from jax import lax
from jax.experimental import pallas as pl
from jax.experimental.pallas import tpu as pltpu
from jax.experimental.pallas import tpu_sc as plsc

_BLK = 8192

_NC = 2
_NS = 16
_NW = _NC * _NS

_SC_PARAMS = pltpu.CompilerParams(needs_layout_passes=False)


def _mlp_body(xt_ref, w1t_ref, b1_ref, w2at_ref, w2bt_ref, b2_ref,
              s_ref, r_ref, g_ref, b_ref):
    xt = xt_ref[...]
    featst = lax.dot_general(
        w1t_ref[...], xt, (((1,), (0,)), ((), ())),
        preferred_element_type=jnp.float32)
    featst = jnp.maximum(featst + b1_ref[...], 0.0)
    out2t = lax.dot_general(
        w2at_ref[...], featst, (((1,), (0,)), ((), ())),
        preferred_element_type=jnp.float32)
    out2t = out2t + lax.dot_general(
        w2bt_ref[...], xt, (((1,), (0,)), ((), ())),
        preferred_element_type=jnp.float32)
    out2t = out2t + b2_ref[...]
    st = jnp.maximum(out2t[0:1, :], 0.0) * xt[6:7, :]
    rgbt = jax.nn.sigmoid(out2t[1:4, :])
    s_ref[...] = st.reshape(_BLK)
    r_ref[...] = rgbt[0:1, :].reshape(_BLK)
    g_ref[...] = rgbt[1:2, :].reshape(_BLK)
    b_ref[...] = rgbt[2:3, :].reshape(_BLK)


def _run_mlp(xt, w1t, b1, w2at, w2bt, b2):
    n = xt.shape[1]
    grid = n // _BLK
    vec = jax.ShapeDtypeStruct((n,), jnp.float32)
    return pl.pallas_call(
        _mlp_body,
        grid=(grid,),
        in_specs=[
            pl.BlockSpec((7, _BLK), lambda i: (0, i)),
            pl.BlockSpec((32, 7), lambda i: (0, 0)),
            pl.BlockSpec((32, 1), lambda i: (0, 0)),
            pl.BlockSpec((4, 32), lambda i: (0, 0)),
            pl.BlockSpec((4, 7), lambda i: (0, 0)),
            pl.BlockSpec((4, 1), lambda i: (0, 0)),
        ],
        out_specs=[pl.BlockSpec((_BLK,), lambda i: (i,))] * 4,
        out_shape=[vec, vec, vec, vec],
    )(xt, w1t, b1, w2at, w2bt, b2)


def _render_body(s_hbm, r_hbm, g_hbm, b_hbm, bg_hbm, out_hbm,
                 s_v, r_v, g_v, b_v, bg_v, out_v):
    wid = lax.axis_index("s") * _NC + lax.axis_index("c")
    spw = (8192 // _NW) * 64
    base = wid * spw
    pltpu.sync_copy(s_hbm.at[pl.ds(base, spw)], s_v)
    pltpu.sync_copy(r_hbm.at[pl.ds(base, spw)], r_v)
    pltpu.sync_copy(g_hbm.at[pl.ds(base, spw)], g_v)
    pltpu.sync_copy(b_hbm.at[pl.ds(base, spw)], b_v)
    pltpu.sync_copy(bg_hbm, bg_v)

    lanes = lax.iota(jnp.int32, 16)
    ray_word = lanes * 64

    def quad(i, carry):
        idx, t, ar, ag, ab, aw = carry
        for _ in range(4):
            s = plsc.load_gather(s_v, [idx])
            r = plsc.load_gather(r_v, [idx])
            g = plsc.load_gather(g_v, [idx])
            b = plsc.load_gather(b_v, [idx])
            e = jnp.exp(-s)
            w = (1.0 - e) * t
            t = t * e
            ar = ar + w * r
            ag = ag + w * g
            ab = ab + w * b
            aw = aw + w
            idx = idx + 1
        return (idx, t, ar, ag, ab, aw)

    for gi in range(8192 // _NW // 16):
        idx0 = ray_word + gi * (16 * 64)
        ones = jnp.ones((16,), jnp.float32)
        zeros = jnp.zeros((16,), jnp.float32)
        _, _, ar, ag, ab, aw = lax.fori_loop(
            0, 16, quad, (idx0, ones, zeros, zeros, zeros, zeros))

        rem = 1.0 - aw
        orow = (gi * 16 + lanes) * 3
        plsc.store_scatter(out_v, [orow], ar + bg_v[pl.ds(0, 16)] * rem)
        plsc.store_scatter(out_v, [orow + 1], ag + bg_v[pl.ds(16, 16)] * rem)
        plsc.store_scatter(out_v, [orow + 2], ab + bg_v[pl.ds(32, 16)] * rem)

    rpw = 8192 // _NW
    pltpu.sync_copy(out_v, out_hbm.at[pl.ds(wid * rpw * 3, rpw * 3)])


def _run_render(s, r, g, b, bg48, n_rays):
    mesh = plsc.VectorSubcoreMesh(core_axis_name="c", subcore_axis_name="s")
    spw = (n_rays // _NW) * 64
    rpw = n_rays // _NW
    kern = pl.kernel(
        _render_body,
        out_type=jax.ShapeDtypeStruct((n_rays * 3,), jnp.float32),
        mesh=mesh,
        scratch_types=[
            pltpu.VMEM((spw,), jnp.float32),
            pltpu.VMEM((spw,), jnp.float32),
            pltpu.VMEM((spw,), jnp.float32),
            pltpu.VMEM((spw,), jnp.float32),
            pltpu.VMEM((48,), jnp.float32),
            pltpu.VMEM((rpw * 3,), jnp.float32),
        ],
        compiler_params=_SC_PARAMS,
    )
    return kern(s, r, g, b, bg48)


@jax.jit
def kernel(packed_samples, packing_info, W_feat, b_feat, W_sigma, b_sigma,
           W_rgb, b_rgb, bg_color):
    n_rays = packing_info.shape[0]
    w1t = jnp.concatenate(
        [W_feat.T, jnp.zeros((32, 4), jnp.float32)], axis=1)
    b1 = b_feat[:, None]
    w2at = jnp.concatenate([W_sigma, W_rgb[:32]], axis=1).T
    w2bt = jnp.zeros((4, 7), jnp.float32)
    w2bt = w2bt.at[1:4, 3:6].set(W_rgb[32:35].T)
    b2 = jnp.concatenate([b_sigma, b_rgb])[:, None]

    xt = packed_samples.T
    s, r, g, b = _run_mlp(xt, w1t, b1, w2at, w2bt, b2)

    bg48 = jnp.repeat(bg_color, 16)
    out = _run_render(s, r, g, b, bg48, n_rays)
    return out.reshape(n_rays, 3)


# speedup vs baseline: 21.2982x; 1.1940x over previous
import jax
import jax.numpy as jnp
from jax import lax
from jax.experimental import pallas as pl
from jax.experimental.pallas import tpu as pltpu
from jax.experimental.pallas import tpu_sc as plsc

_BLK = 16384

_NC = 2
_NS = 16
_NW = _NC * _NS

_SC_PARAMS = pltpu.CompilerParams(needs_layout_passes=False)


def _mlp_body(xt_ref, w1t_ref, b1_ref, w2at_ref, w2bt_ref, b2_ref,
              s_ref, r_ref, g_ref, b_ref):
    xt = xt_ref[...]
    featst = lax.dot_general(
        w1t_ref[...], xt, (((1,), (0,)), ((), ())),
        preferred_element_type=jnp.float32)
    featst = jnp.maximum(featst + b1_ref[...], 0.0)
    out2t = lax.dot_general(
        w2at_ref[...], featst, (((1,), (0,)), ((), ())),
        preferred_element_type=jnp.float32)
    out2t = out2t + lax.dot_general(
        w2bt_ref[...], xt, (((1,), (0,)), ((), ())),
        preferred_element_type=jnp.float32)
    out2t = out2t + b2_ref[...]
    st = jnp.maximum(out2t[0:1, :], 0.0) * (-xt[6:7, :])
    rgbt = jax.nn.sigmoid(out2t[1:4, :])
    s_ref[...] = st.reshape(_BLK)
    r_ref[...] = rgbt[0:1, :].reshape(_BLK)
    g_ref[...] = rgbt[1:2, :].reshape(_BLK)
    b_ref[...] = rgbt[2:3, :].reshape(_BLK)


def _run_mlp(xt, w1t, b1, w2at, w2bt, b2):
    n = xt.shape[1]
    grid = n // _BLK
    vec = jax.ShapeDtypeStruct((n,), jnp.float32)
    return pl.pallas_call(
        _mlp_body,
        grid=(grid,),
        in_specs=[
            pl.BlockSpec((7, _BLK), lambda i: (0, i)),
            pl.BlockSpec((32, 7), lambda i: (0, 0)),
            pl.BlockSpec((32, 1), lambda i: (0, 0)),
            pl.BlockSpec((4, 32), lambda i: (0, 0)),
            pl.BlockSpec((4, 7), lambda i: (0, 0)),
            pl.BlockSpec((4, 1), lambda i: (0, 0)),
        ],
        out_specs=[pl.BlockSpec((_BLK,), lambda i: (i,))] * 4,
        out_shape=[vec, vec, vec, vec],
    )(xt, w1t, b1, w2at, w2bt, b2)


def _render_body(s_hbm, r_hbm, g_hbm, b_hbm, bg_hbm, out_hbm,
                 s_v, r_v, g_v, b_v, bg_v, out_v, sem):
    wid = lax.axis_index("s") * _NC + lax.axis_index("c")
    spw = (8192 // _NW) * 64
    base = wid * spw
    c1 = pltpu.make_async_copy(s_hbm.at[pl.ds(base, spw)], s_v, sem)
    c2 = pltpu.make_async_copy(r_hbm.at[pl.ds(base, spw)], r_v, sem)
    c3 = pltpu.make_async_copy(g_hbm.at[pl.ds(base, spw)], g_v, sem)
    c4 = pltpu.make_async_copy(b_hbm.at[pl.ds(base, spw)], b_v, sem)
    c1.start(); c2.start(); c3.start(); c4.start()
    pltpu.sync_copy(bg_hbm, bg_v)
    c1.wait(); c2.wait(); c3.wait(); c4.wait()

    lanes = lax.iota(jnp.int32, 16)
    ray_word = lanes * 64

    def group(gi, _):
        idx0 = ray_word + gi * (16 * 64)
        t = jnp.ones((16,), jnp.float32)
        zeros = jnp.zeros((16,), jnp.float32)
        ar, ag, ab, aw = zeros, zeros, zeros, zeros
        for i in range(64):
            idx = idx0 + i
            e = jnp.exp(plsc.load_gather(s_v, [idx]))
            r = plsc.load_gather(r_v, [idx])
            g = plsc.load_gather(g_v, [idx])
            b = plsc.load_gather(b_v, [idx])
            w = (1.0 - e) * t
            t = t * e
            ar = ar + w * r
            ag = ag + w * g
            ab = ab + w * b
            aw = aw + w

        rem = 1.0 - aw
        orow = (gi * 16 + lanes) * 3
        plsc.store_scatter(out_v, [orow], ar + bg_v[pl.ds(0, 16)] * rem)
        plsc.store_scatter(out_v, [orow + 1], ag + bg_v[pl.ds(16, 16)] * rem)
        plsc.store_scatter(out_v, [orow + 2], ab + bg_v[pl.ds(32, 16)] * rem)
        return 0

    lax.fori_loop(0, 8192 // _NW // 16, group, 0)

    rpw = 8192 // _NW
    pltpu.sync_copy(out_v, out_hbm.at[pl.ds(wid * rpw * 3, rpw * 3)])


def _run_render(s, r, g, b, bg48, n_rays):
    mesh = plsc.VectorSubcoreMesh(core_axis_name="c", subcore_axis_name="s")
    spw = (n_rays // _NW) * 64
    rpw = n_rays // _NW
    kern = pl.kernel(
        _render_body,
        out_type=jax.ShapeDtypeStruct((n_rays * 3,), jnp.float32),
        mesh=mesh,
        scratch_types=[
            pltpu.VMEM((spw,), jnp.float32),
            pltpu.VMEM((spw,), jnp.float32),
            pltpu.VMEM((spw,), jnp.float32),
            pltpu.VMEM((spw,), jnp.float32),
            pltpu.VMEM((48,), jnp.float32),
            pltpu.VMEM((rpw * 3,), jnp.float32),
            pltpu.SemaphoreType.DMA,
        ],
        compiler_params=_SC_PARAMS,
    )
    return kern(s, r, g, b, bg48)


@jax.jit
def kernel(packed_samples, packing_info, W_feat, b_feat, W_sigma, b_sigma,
           W_rgb, b_rgb, bg_color):
    n_rays = packing_info.shape[0]
    w1t = jnp.concatenate(
        [W_feat.T, jnp.zeros((32, 4), jnp.float32)], axis=1)
    b1 = b_feat[:, None]
    w2at = jnp.concatenate([W_sigma, W_rgb[:32]], axis=1).T
    w2bt = jnp.zeros((4, 7), jnp.float32)
    w2bt = w2bt.at[1:4, 3:6].set(W_rgb[32:35].T)
    b2 = jnp.concatenate([b_sigma, b_rgb])[:, None]

    xt = packed_samples.T
    s, r, g, b = _run_mlp(xt, w1t, b1, w2at, w2bt, b2)

    bg48 = jnp.repeat(bg_color, 16)
    out = _run_render(s, r, g, b, bg48, n_rays)
    return out.reshape(n_rays, 3)
